# baseline (device time: 34934 ns/iter reference)
import functools

import jax
import jax.numpy as jnp
from jax import lax
from jax.experimental import pallas as pl
from jax.experimental.pallas import tpu as pltpu

N_DEV = 4
N_EXPERTS = 16
N_LOCAL_E = 4
N_TOK = 1024
D_IN = 512
D_OUT = 1024
BLK = N_TOK // N_DEV
DISTS = (2, 1, 3, 0)


def kernel(x, router_W, route_idx, expert_W):
    def body(x_ref, rw_ref, idx_ref, ewj_ref, out_ref,
             coeff_ref, acc_ref, send_buf, recv_buf, send_sems, recv_sems):
        j = pl.program_id(0)
        b = pl.program_id(1)
        my_pos = lax.axis_index("i")
        barrier_sem = pltpu.get_barrier_semaphore()

        @pl.when((j == 0) & (b == 0))
        def _prologue():
            for d in range(1, N_DEV):
                nbr = lax.rem(my_pos + d, N_DEV)
                pl.semaphore_signal(
                    barrier_sem, inc=1,
                    device_id=(nbr,), device_id_type=pl.DeviceIdType.MESH,
                )
            scores = jnp.dot(x_ref[:, :], rw_ref[:, :],
                             preferred_element_type=jnp.float32)
            m = jnp.max(scores, axis=1, keepdims=True)
            p = jnp.exp(scores - m)
            p = p / jnp.sum(p, axis=1, keepdims=True)
            idx0 = idx_ref[:, 0:1]
            idx1 = idx_ref[:, 1:2]
            iota = lax.broadcasted_iota(jnp.int32, (N_TOK, N_EXPERTS), 1)
            g0 = jnp.sum(jnp.where(iota == idx0, p, 0.0), axis=1,
                         keepdims=True)
            g1 = jnp.sum(jnp.where(iota == idx1, p, 0.0), axis=1,
                         keepdims=True)
            gs = g0 + g1
            g0n = g0 / gs
            g1n = g1 / gs
            base = my_pos * N_LOCAL_E
            for jj in range(N_LOCAL_E):
                e = base + jj
                coeff_ref[:, jj:jj + 1] = (jnp.where(idx0 == e, g0n, 0.0)
                                           + jnp.where(idx1 == e, g1n, 0.0))

        dsel = jnp.where(b == 0, 2, jnp.where(b == 1, 1,
                                              jnp.where(b == 2, 3, 0)))
        owner = lax.rem(my_pos + dsel, N_DEV)
        rows = pl.ds(owner * BLK, BLK)

        xb = x_ref[rows, :].astype(jnp.bfloat16)
        cfull = coeff_ref[rows, :]
        jiota = lax.broadcasted_iota(jnp.int32, (BLK, N_LOCAL_E), 1)
        c = jnp.sum(jnp.where(jiota == j, cfull, 0.0), axis=1, keepdims=True)
        contrib = c * jnp.dot(xb, ewj_ref[0].astype(jnp.bfloat16),
                              preferred_element_type=jnp.float32)

        @pl.when(j == 0)
        def _init():
            acc_ref[rows, :] = contrib

        @pl.when(j > 0)
        def _accum():
            acc_ref[rows, :] = acc_ref[rows, :] + contrib

        last_j = j == N_LOCAL_E - 1

        @pl.when(last_j & (b == 0))
        def _barrier_wait():
            pl.semaphore_wait(barrier_sem, N_DEV - 1)

        def make_rdma(d):
            slot = 3 - d
            tgt = lax.rem(my_pos + d, N_DEV)
            return pltpu.make_async_remote_copy(
                src_ref=send_buf.at[slot],
                dst_ref=recv_buf.at[slot],
                send_sem=send_sems.at[slot],
                recv_sem=recv_sems.at[slot],
                device_id=(tgt,),
                device_id_type=pl.DeviceIdType.MESH,
            )

        for k, d in enumerate(DISTS[:3]):
            @pl.when(last_j & (b == k))
            def _send(d=d):
                ow = lax.rem(my_pos + d, N_DEV)
                send_buf[3 - d] = acc_ref[
                    pl.ds(ow * BLK, BLK), :].astype(jnp.bfloat16)
                make_rdma(d).start()

        @pl.when(last_j & (b == 3))
        def _epilogue():
            rdmas = [make_rdma(d) for d in DISTS[:3]]
            for rdma in rdmas:
                rdma.wait_recv()
            out_ref[:, :] = (acc_ref[pl.ds(my_pos * BLK, BLK), :]
                             + recv_buf[0].astype(jnp.float32)
                             + recv_buf[1].astype(jnp.float32)
                             + recv_buf[2].astype(jnp.float32))
            for rdma in rdmas:
                rdma.wait_send()

    return pl.pallas_call(
        body,
        grid=(N_LOCAL_E, N_DEV),
        out_shape=jax.ShapeDtypeStruct((BLK, D_OUT), jnp.float32),
        in_specs=[
            pl.BlockSpec((N_TOK, D_IN), lambda j, b: (0, 0)),
            pl.BlockSpec((D_IN, N_EXPERTS), lambda j, b: (0, 0)),
            pl.BlockSpec((N_TOK, 2), lambda j, b: (0, 0)),
            pl.BlockSpec((1, D_IN, D_OUT), lambda j, b: (j, 0, 0)),
        ],
        out_specs=pl.BlockSpec((BLK, D_OUT), lambda j, b: (0, 0)),
        scratch_shapes=[
            pltpu.VMEM((N_TOK, N_LOCAL_E), jnp.float32),
            pltpu.VMEM((N_TOK, D_OUT), jnp.float32),
            pltpu.VMEM((N_DEV - 1, BLK, D_OUT), jnp.bfloat16),
            pltpu.VMEM((N_DEV - 1, BLK, D_OUT), jnp.bfloat16),
            pltpu.SemaphoreType.DMA((N_DEV - 1,)),
            pltpu.SemaphoreType.DMA((N_DEV - 1,)),
        ],
        compiler_params=pltpu.CompilerParams(
            collective_id=0,
            dimension_semantics=("arbitrary", "arbitrary"),
        ),
    )(x, router_W, route_idx, expert_W)


# device time: 26862 ns/iter; 1.3005x vs baseline; 1.3005x over previous
import jax
import jax.numpy as jnp
from jax import lax
from jax.experimental import pallas as pl
from jax.experimental.pallas import tpu as pltpu

N_DEV = 4
N_EXPERTS = 16
N_LOCAL_E = 4
N_TOK = 1024
D_IN = 512
D_OUT = 1024
BLK = N_TOK // N_DEV


def kernel(x, router_W, route_idx, expert_W):
    def body(x_ref, rw_ref, idx_ref, ew_ref, out_ref,
             coeff_ref, send_buf, recv_buf, send_sems, recv_sems):
        my_pos = lax.axis_index("i")

        barrier_sem = pltpu.get_barrier_semaphore()
        for d in range(1, N_DEV):
            nbr = lax.rem(my_pos + d, N_DEV)
            pl.semaphore_signal(
                barrier_sem, inc=1,
                device_id=(nbr,), device_id_type=pl.DeviceIdType.MESH,
            )
        pl.semaphore_wait(barrier_sem, N_DEV - 1)

        scores = jnp.dot(x_ref[:, :], rw_ref[:, :],
                         preferred_element_type=jnp.float32)
        m = jnp.max(scores, axis=1, keepdims=True)
        p = jnp.exp(scores - m)
        p = p / jnp.sum(p, axis=1, keepdims=True)

        idx0 = idx_ref[:, 0:1]
        idx1 = idx_ref[:, 1:2]
        iota = lax.broadcasted_iota(jnp.int32, (N_TOK, N_EXPERTS), 1)
        g0 = jnp.sum(jnp.where(iota == idx0, p, 0.0), axis=1, keepdims=True)
        g1 = jnp.sum(jnp.where(iota == idx1, p, 0.0), axis=1, keepdims=True)
        gs = g0 + g1
        g0 = g0 / gs
        g1 = g1 / gs

        base = my_pos * N_LOCAL_E
        for j in range(N_LOCAL_E):
            e = base + j
            coeff_ref[:, j:j + 1] = (jnp.where(idx0 == e, g0, 0.0)
                                     + jnp.where(idx1 == e, g1, 0.0))

        def partial_block(b):
            xb = x_ref[pl.ds(b * BLK, BLK), :].astype(jnp.bfloat16)
            acc = jnp.zeros((BLK, D_OUT), jnp.float32)
            for j in range(N_LOCAL_E):
                c = coeff_ref[pl.ds(b * BLK, BLK), j:j + 1]
                acc = acc + c * jnp.dot(xb, ew_ref[j].astype(jnp.bfloat16),
                                        preferred_element_type=jnp.float32)
            return acc

        rdmas = []
        for d in (2, 1, 3):
            owner = lax.rem(my_pos + d, N_DEV)
            slot = 3 - d
            send_buf[slot] = partial_block(owner).astype(jnp.bfloat16)
            rdma = pltpu.make_async_remote_copy(
                src_ref=send_buf.at[slot],
                dst_ref=recv_buf.at[slot],
                send_sem=send_sems.at[slot],
                recv_sem=recv_sems.at[slot],
                device_id=(owner,),
                device_id_type=pl.DeviceIdType.MESH,
            )
            rdma.start()
            rdmas.append(rdma)

        own = partial_block(my_pos)

        for rdma in rdmas:
            rdma.wait_recv()
        out_ref[:, :] = (own
                         + recv_buf[0].astype(jnp.float32)
                         + recv_buf[1].astype(jnp.float32)
                         + recv_buf[2].astype(jnp.float32))
        for rdma in rdmas:
            rdma.wait_send()

    return pl.pallas_call(
        body,
        out_shape=jax.ShapeDtypeStruct((BLK, D_OUT), jnp.float32),
        in_specs=[
            pl.BlockSpec(memory_space=pltpu.VMEM),
            pl.BlockSpec(memory_space=pltpu.VMEM),
            pl.BlockSpec(memory_space=pltpu.VMEM),
            pl.BlockSpec(memory_space=pltpu.VMEM),
        ],
        out_specs=pl.BlockSpec(memory_space=pltpu.VMEM),
        scratch_shapes=[
            pltpu.VMEM((N_TOK, N_LOCAL_E), jnp.float32),
            pltpu.VMEM((N_DEV - 1, BLK, D_OUT), jnp.bfloat16),
            pltpu.VMEM((N_DEV - 1, BLK, D_OUT), jnp.bfloat16),
            pltpu.SemaphoreType.DMA((N_DEV - 1,)),
            pltpu.SemaphoreType.DMA((N_DEV - 1,)),
        ],
        compiler_params=pltpu.CompilerParams(collective_id=0),
    )(x, router_W, route_idx, expert_W)
